# Initial kernel scaffold; baseline (speedup 1.0000x reference)
import jax
import jax.numpy as jnp
from jax.experimental import pallas as pl


def kernel(tokens, table):
    raise NotImplementedError("write your pallas kernel here")


# speedup vs baseline: 1.4589x; 1.4589x over previous
import functools

import jax
import jax.numpy as jnp
from jax import lax
from jax.experimental import pallas as pl
from jax.experimental.pallas import tpu as pltpu
from jax.experimental.pallas import tpu_sc as plsc

_D = 32
_NW = 32
_CHUNK = 1024
_GRP = 128


@functools.cache
def _make_gather(n_rows: int, d: int):
    per_w = n_rows // _NW
    n_chunks = per_w // _CHUNK
    mesh = plsc.VectorSubcoreMesh(core_axis_name="c", subcore_axis_name="s")

    @functools.partial(
        pl.kernel,
        mesh=mesh,
        out_type=jax.ShapeDtypeStruct((n_rows, d), jnp.float32),
        scratch_types=[
            pltpu.VMEM((_CHUNK,), jnp.int32),
            pltpu.VMEM((_CHUNK, d), jnp.float32),
            pltpu.SemaphoreType.DMA,
        ],
        compiler_params=pltpu.CompilerParams(use_tc_tiling_on_sc=False),
    )
    def body(tokens_hbm, table_hbm, out_hbm, idx_v, rows_v, sem):
        wid = lax.axis_index("s") * 2 + lax.axis_index("c")
        base = wid * per_w

        def step(ci, carry):
            off = base + ci * _CHUNK
            pltpu.sync_copy(tokens_hbm.at[pl.ds(off, _CHUNK)], idx_v)
            copies = [
                pltpu.async_copy(
                    table_hbm.at[idx_v.at[pl.ds(g * _GRP, _GRP)]],
                    rows_v.at[pl.ds(g * _GRP, _GRP)],
                    sem,
                )
                for g in range(_CHUNK // _GRP)
            ]
            for c in copies:
                c.wait()
            pltpu.sync_copy(rows_v, out_hbm.at[pl.ds(off, _CHUNK)])
            return carry

        lax.fori_loop(0, n_chunks, step, 0)

    return body


def kernel(tokens, table):
    b, h = tokens.shape
    d = table.shape[1]
    flat = tokens.reshape(-1).astype(jnp.int32)
    out = _make_gather(flat.shape[0], d)(flat, table)
    return out.reshape(b, h, d)


# speedup vs baseline: 1.4703x; 1.0078x over previous
import functools

import jax
import jax.numpy as jnp
from jax import lax
from jax.experimental import pallas as pl
from jax.experimental.pallas import tpu as pltpu
from jax.experimental.pallas import tpu_sc as plsc

_D = 32
_NW = 32
_CHUNK = 512
_GRP = 128


@functools.cache
def _make_gather(n_rows: int, d: int):
    per_w = n_rows // _NW
    n_chunks = per_w // _CHUNK
    mesh = plsc.VectorSubcoreMesh(core_axis_name="c", subcore_axis_name="s")

    @functools.partial(
        pl.kernel,
        mesh=mesh,
        out_type=jax.ShapeDtypeStruct((n_rows, d), jnp.float32),
        scratch_types=[
            pltpu.VMEM((_CHUNK,), jnp.int32),
            pltpu.VMEM((_CHUNK,), jnp.int32),
            pltpu.VMEM((_CHUNK, d), jnp.float32),
            pltpu.VMEM((_CHUNK, d), jnp.float32),
            pltpu.SemaphoreType.DMA,
            pltpu.SemaphoreType.DMA,
            pltpu.SemaphoreType.DMA,
            pltpu.SemaphoreType.DMA,
        ],
        compiler_params=pltpu.CompilerParams(use_tc_tiling_on_sc=False),
    )
    def body(tokens_hbm, table_hbm, out_hbm, idx_v, idx_b, rows_v, rows_b, sem, semb, osem, osemb):
        wid = lax.axis_index("s") * 2 + lax.axis_index("c")
        base = wid * per_w
        idx = (idx_v, idx_b)
        rows = (rows_v, rows_b)
        gsem = (sem, semb)
        wsem = (osem, osemb)
        n_pairs = n_chunks // 2

        def fire(ci, b):
            off = base + ci * _CHUNK
            pltpu.sync_copy(tokens_hbm.at[pl.ds(off, _CHUNK)], idx[b])
            for g in range(_CHUNK // _GRP):
                pltpu.async_copy(
                    table_hbm.at[idx[b].at[pl.ds(g * _GRP, _GRP)]],
                    rows[b].at[pl.ds(g * _GRP, _GRP)],
                    gsem[b],
                )

        def drain_and_write(ci, b):
            for g in range(_CHUNK // _GRP):
                pltpu.make_async_copy(
                    table_hbm.at[idx[b].at[pl.ds(g * _GRP, _GRP)]],
                    rows[b].at[pl.ds(g * _GRP, _GRP)],
                    gsem[b],
                ).wait()
            off = base + ci * _CHUNK
            pltpu.async_copy(rows[b], out_hbm.at[pl.ds(off, _CHUNK)], wsem[b])

        def wait_write(ci, b):
            off = base + ci * _CHUNK
            pltpu.make_async_copy(
                rows[b], out_hbm.at[pl.ds(off, _CHUNK)], wsem[b]
            ).wait()

        fire(0, 0)

        def pair(cj, carry):
            i = 2 * cj + 1

            @pl.when(cj > 0)
            def _():
                wait_write(i - 2, 1)

            fire(i, 1)
            drain_and_write(i - 1, 0)

            @pl.when(cj < n_pairs - 1)
            def _():
                wait_write(i - 1, 0)
                fire(i + 1, 0)

            drain_and_write(i, 1)
            return carry

        lax.fori_loop(0, n_pairs, pair, 0)
        wait_write(n_chunks - 2, 0)
        wait_write(n_chunks - 1, 1)

    return body


def kernel(tokens, table):
    b, h = tokens.shape
    d = table.shape[1]
    flat = tokens.reshape(-1).astype(jnp.int32)
    out = _make_gather(flat.shape[0], d)(flat, table)
    return out.reshape(b, h, d)


# speedup vs baseline: 1.9880x; 1.3521x over previous
import functools

import jax
import jax.numpy as jnp
from jax import lax
from jax.experimental import pallas as pl
from jax.experimental.pallas import tpu as pltpu
from jax.experimental.pallas import tpu_sc as plsc

_D = 32
_NW = 32
_CHUNK = 512
_GRP = 128


@functools.cache
def _make_gather(n_rows: int, d: int):
    per_w = n_rows // _NW
    n_chunks = per_w // _CHUNK
    mesh = plsc.VectorSubcoreMesh(core_axis_name="c", subcore_axis_name="s")

    @functools.partial(
        pl.kernel,
        mesh=mesh,
        out_type=jax.ShapeDtypeStruct((n_rows, 128), jnp.float32),
        scratch_types=[
            pltpu.VMEM((_CHUNK,), jnp.int32),
            pltpu.VMEM((_CHUNK,), jnp.int32),
            pltpu.VMEM((_CHUNK, d), jnp.float32),
            pltpu.VMEM((_CHUNK, d), jnp.float32),
            pltpu.SemaphoreType.DMA,
            pltpu.SemaphoreType.DMA,
            pltpu.SemaphoreType.DMA,
            pltpu.SemaphoreType.DMA,
        ],
        compiler_params=pltpu.CompilerParams(use_tc_tiling_on_sc=False),
    )
    def body(tokens_hbm, table_hbm, out_hbm, idx_v, idx_b, rows_v, rows_b, sem, semb, osem, osemb):
        wid = lax.axis_index("s") * 2 + lax.axis_index("c")
        base = wid * per_w
        idx = (idx_v, idx_b)
        rows = (rows_v, rows_b)
        gsem = (sem, semb)
        wsem = (osem, osemb)
        n_pairs = n_chunks // 2

        def fire(ci, b):
            off = base + ci * _CHUNK
            pltpu.sync_copy(tokens_hbm.at[pl.ds(off, _CHUNK)], idx[b])
            for g in range(_CHUNK // _GRP):
                pltpu.async_copy(
                    table_hbm.at[idx[b].at[pl.ds(g * _GRP, _GRP)]],
                    rows[b].at[pl.ds(g * _GRP, _GRP)],
                    gsem[b],
                )

        def drain_and_write(ci, b):
            for g in range(_CHUNK // _GRP):
                pltpu.make_async_copy(
                    table_hbm.at[idx[b].at[pl.ds(g * _GRP, _GRP)]],
                    rows[b].at[pl.ds(g * _GRP, _GRP)],
                    gsem[b],
                ).wait()
            off = base + ci * _CHUNK
            pltpu.async_copy(
                rows[b], out_hbm.at[pl.ds(off, _CHUNK), pl.ds(0, d)], wsem[b]
            )

        def wait_write(ci, b):
            off = base + ci * _CHUNK
            pltpu.make_async_copy(
                rows[b], out_hbm.at[pl.ds(off, _CHUNK), pl.ds(0, d)], wsem[b]
            ).wait()

        fire(0, 0)

        def pair(cj, carry):
            i = 2 * cj + 1

            @pl.when(cj > 0)
            def _():
                wait_write(i - 2, 1)

            fire(i, 1)
            drain_and_write(i - 1, 0)

            @pl.when(cj < n_pairs - 1)
            def _():
                wait_write(i - 1, 0)
                fire(i + 1, 0)

            drain_and_write(i, 1)
            return carry

        lax.fori_loop(0, n_pairs, pair, 0)
        wait_write(n_chunks - 2, 0)
        wait_write(n_chunks - 1, 1)

    return body


def kernel(tokens, table):
    b, h = tokens.shape
    d = table.shape[1]
    flat = tokens.reshape(-1).astype(jnp.int32)
    out = _make_gather(flat.shape[0], d)(flat, table)
    return out[:, :d].reshape(b, h, d)
